# baseline (speedup 1.0000x reference)
import functools

import jax
import jax.numpy as jnp
from jax import lax
from jax.experimental import pallas as pl
from jax.experimental.pallas import tpu as pltpu
from jax.experimental.pallas import tpu_sc as plsc

N = 10000
D = 128
BLK = 1000
GRID = N // BLK
NEG_INF = -3.0e38

E = 320000
NW = 32
NSUB = 16
CH = 80
E2 = 327680
ECH = E2 // CH
NCH = E2 // NW // CH
NCH2 = E2 // NSUB // CH
NPAD = 10240
TROW = NPAD // NSUB


def _lrelu(x):
    return jnp.where(x > 0, x, 0.2 * x)


def _tc1_body(xt_ref, xf_ref, w1_ref, wg_ref, al_ref, ar_ref,
              h1_ref, hg_ref, asrc_ref, adst_ref, m_ref):
    i = pl.program_id(0)
    h1 = jnp.dot(xt_ref[...], w1_ref[...], preferred_element_type=jnp.float32)
    h1_ref[...] = h1
    hg = jnp.dot(xf_ref[...], wg_ref[...], preferred_element_type=jnp.float32)
    hg_ref[...] = hg
    asrc = jnp.sum(hg * al_ref[...], axis=-1)
    adst = jnp.sum(hg * ar_ref[...], axis=-1)
    asrc_ref[...] = asrc[:, None]
    adst_ref[...] = adst[:, None]

    cur = jnp.where(i == 0, jnp.full((1, 2), NEG_INF, jnp.float32), m_ref[...])
    new = jnp.stack([jnp.max(asrc), jnp.max(adst)]).reshape(1, 2)
    m_ref[...] = jnp.maximum(cur, new)


def _tc_stage1(topo_x, feat_x, W1, Wg, att_l, att_r):
    return pl.pallas_call(
        _tc1_body,
        grid=(GRID,),
        in_specs=[
            pl.BlockSpec((BLK, D), lambda i: (i, 0)),
            pl.BlockSpec((BLK, D), lambda i: (i, 0)),
            pl.BlockSpec((D, D), lambda i: (0, 0)),
            pl.BlockSpec((D, D), lambda i: (0, 0)),
            pl.BlockSpec((1, D), lambda i: (0, 0)),
            pl.BlockSpec((1, D), lambda i: (0, 0)),
        ],
        out_specs=[
            pl.BlockSpec((BLK, D), lambda i: (i, 0)),
            pl.BlockSpec((BLK, D), lambda i: (i, 0)),
            pl.BlockSpec((BLK, 1), lambda i: (i, 0)),
            pl.BlockSpec((BLK, 1), lambda i: (i, 0)),
            pl.BlockSpec((1, 2), lambda i: (0, 0)),
        ],
        out_shape=[
            jax.ShapeDtypeStruct((N, D), jnp.float32),
            jax.ShapeDtypeStruct((N, D), jnp.float32),
            jax.ShapeDtypeStruct((N, 1), jnp.float32),
            jax.ShapeDtypeStruct((N, 1), jnp.float32),
            jax.ShapeDtypeStruct((1, 2), jnp.float32),
        ],
    )(topo_x, feat_x, W1, Wg, att_l.reshape(1, D), att_r.reshape(1, D))


def _tc2_body(d0_ref, d1_ref, h1_ref, dinv_ref, hs1_ref):
    deg = d0_ref[:, 0] + d1_ref[:, 0] + 1.0
    dinv = lax.rsqrt(deg)
    dinv_ref[...] = dinv[:, None]
    hs1_ref[...] = h1_ref[...] * dinv[:, None]


def _tc_stage2(deg0, deg1, h1):
    return pl.pallas_call(
        _tc2_body,
        grid=(GRID,),
        in_specs=[
            pl.BlockSpec((BLK, 1), lambda i: (i, 0)),
            pl.BlockSpec((BLK, 1), lambda i: (i, 0)),
            pl.BlockSpec((BLK, D), lambda i: (i, 0)),
        ],
        out_specs=[
            pl.BlockSpec((BLK, 1), lambda i: (i, 0)),
            pl.BlockSpec((BLK, D), lambda i: (i, 0)),
        ],
        out_shape=[
            jax.ShapeDtypeStruct((N, 1), jnp.float32),
            jax.ShapeDtypeStruct((N, D), jnp.float32),
        ],
    )(deg0, deg1, h1)


def _tc3_body(a0_ref, a1_ref, h1_ref, dinv_ref, b1_ref, w2_ref,
              h2_ref, hs2_ref):
    di = dinv_ref[:, 0]
    a = a0_ref[...] + a1_ref[...]
    x2 = jnp.maximum(
        di[:, None] * a + (di * di)[:, None] * h1_ref[...] + b1_ref[...], 0.0)
    h2 = jnp.dot(x2, w2_ref[...], preferred_element_type=jnp.float32)
    h2_ref[...] = h2
    hs2_ref[...] = h2 * di[:, None]


def _tc_stage3(acc1a, acc1b, h1, dinv, b1, W2):
    return pl.pallas_call(
        _tc3_body,
        grid=(GRID,),
        in_specs=[
            pl.BlockSpec((BLK, D), lambda i: (i, 0)),
            pl.BlockSpec((BLK, D), lambda i: (i, 0)),
            pl.BlockSpec((BLK, D), lambda i: (i, 0)),
            pl.BlockSpec((BLK, 1), lambda i: (i, 0)),
            pl.BlockSpec((1, D), lambda i: (0, 0)),
            pl.BlockSpec((D, D), lambda i: (0, 0)),
        ],
        out_specs=[
            pl.BlockSpec((BLK, D), lambda i: (i, 0)),
            pl.BlockSpec((BLK, D), lambda i: (i, 0)),
        ],
        out_shape=[
            jax.ShapeDtypeStruct((N, D), jnp.float32),
            jax.ShapeDtypeStruct((N, D), jnp.float32),
        ],
    )(acc1a, acc1b, h1, dinv, b1.reshape(1, D), W2)


def _tc4_body(a20_ref, a21_ref, ag0_ref, ag1_ref, h2_ref, hg_ref, dinv_ref,
              s0_ref, s1_ref, asrc_ref, adst_ref, m_ref, b2_ref, wd_ref,
              res_ref):
    di = dinv_ref[:, 0]
    a2 = a20_ref[...] + a21_ref[...]
    topo_z = (di[:, None] * a2 + (di * di)[:, None] * h2_ref[...]
              + b2_ref[...])
    mv = m_ref[...]
    M = _lrelu(mv[0, 0] + mv[0, 1])
    ex_self = jnp.exp(_lrelu(asrc_ref[:, 0] + adst_ref[:, 0]) - M)
    s = s0_ref[:, 0] + s1_ref[:, 0] + ex_self
    ag = ag0_ref[...] + ag1_ref[...]
    feat_z = (ag + ex_self[:, None] * hg_ref[...]) / (s[:, None] + 1e-16)
    fzw = jnp.dot(feat_z, wd_ref[...], preferred_element_type=jnp.float32)
    res = jax.nn.sigmoid(jnp.sum(topo_z * fzw, axis=-1))
    res_ref[...] = res[:, None]


def _tc_stage4(a20, a21, ag0, ag1, h2, hg, dinv, s0, s1, asrc, adst, mm,
               b2, Wd):
    res = pl.pallas_call(
        _tc4_body,
        grid=(GRID,),
        in_specs=[
            pl.BlockSpec((BLK, D), lambda i: (i, 0)),
            pl.BlockSpec((BLK, D), lambda i: (i, 0)),
            pl.BlockSpec((BLK, D), lambda i: (i, 0)),
            pl.BlockSpec((BLK, D), lambda i: (i, 0)),
            pl.BlockSpec((BLK, D), lambda i: (i, 0)),
            pl.BlockSpec((BLK, D), lambda i: (i, 0)),
            pl.BlockSpec((BLK, 1), lambda i: (i, 0)),
            pl.BlockSpec((BLK, 1), lambda i: (i, 0)),
            pl.BlockSpec((BLK, 1), lambda i: (i, 0)),
            pl.BlockSpec((BLK, 1), lambda i: (i, 0)),
            pl.BlockSpec((BLK, 1), lambda i: (i, 0)),
            pl.BlockSpec((1, 2), lambda i: (0, 0)),
            pl.BlockSpec((1, D), lambda i: (0, 0)),
            pl.BlockSpec((D, D), lambda i: (0, 0)),
        ],
        out_specs=[pl.BlockSpec((BLK, 1), lambda i: (i, 0))],
        out_shape=[jax.ShapeDtypeStruct((N, 1), jnp.float32)],
    )(a20, a21, ag0, ag1, h2, hg, dinv, s0, s1, asrc, adst, mm,
      b2.reshape(1, D), Wd)[0]
    return res.reshape(N)


_SC_MESH = plsc.VectorSubcoreMesh(core_axis_name="c", subcore_axis_name="s")


def _sc1_body(td_hbm, fs_hbm, fd_hbm, asrc_hbm, adst_hbm, m_hbm,
              deg_out, s_out, ex_out,
              asrc_v, adst_v, tdi, fsi, fdi, exv, ones_v, zv, m_v,
              deg_sh, s_sh):
    cid = lax.axis_index("c")
    sid = lax.axis_index("s")
    wid = cid * NSUB + sid

    def _zbody(i, _):
        zv[pl.ds(i * 16, 16)] = jnp.zeros((16,), jnp.float32)
        return _
    lax.fori_loop(0, TROW // 16, _zbody, None)
    pltpu.sync_copy(zv, deg_sh.at[pl.ds(sid * TROW, TROW)])
    pltpu.sync_copy(zv, s_sh.at[pl.ds(sid * TROW, TROW)])

    def _obody(i, _):
        ones_v[pl.ds(i * 16, 16)] = jnp.ones((16,), jnp.float32)
        return _
    lax.fori_loop(0, CH // 16, _obody, None)

    pltpu.sync_copy(asrc_hbm, asrc_v)
    pltpu.sync_copy(adst_hbm, adst_v)
    pltpu.sync_copy(m_hbm, m_v)
    pltpu.sync_copy(td_hbm.at[pl.ds(wid * NCH, NCH)], tdi)
    pltpu.sync_copy(fs_hbm.at[pl.ds(wid * NCH, NCH)], fsi)
    pltpu.sync_copy(fd_hbm.at[pl.ds(wid * NCH, NCH)], fdi)
    plsc.subcore_barrier()

    mv = m_v[pl.ds(0, 16)]
    M = _lrelu(mv[0] + mv[1])

    def _chunk(j, _):
        for r in range(CH // 16):
            sv = fsi[j, pl.ds(r * 16, 16)]
            dv = fdi[j, pl.ds(r * 16, 16)]
            av = plsc.load_gather(asrc_v, [sv])
            bv = plsc.load_gather(adst_v, [dv])
            ex = jnp.exp(_lrelu(av + bv) - M)
            exv[j, pl.ds(r * 16, 16)] = ex
        pltpu.sync_copy(exv.at[j], s_sh.at[fdi.at[j]], add=True)
        pltpu.sync_copy(ones_v, deg_sh.at[tdi.at[j]], add=True)
        return _
    lax.fori_loop(0, NCH, _chunk, None)

    pltpu.sync_copy(exv, ex_out.at[pl.ds(wid * NCH, NCH)])
    plsc.subcore_barrier()

    base = cid * NPAD + sid * TROW
    pltpu.sync_copy(deg_sh.at[pl.ds(sid * TROW, TROW)],
                    deg_out.at[pl.ds(base, TROW)])
    pltpu.sync_copy(s_sh.at[pl.ds(sid * TROW, TROW)],
                    s_out.at[pl.ds(base, TROW)])


def _sc_stage1(td2, fs2, fd2, asrc, adst, m16):
    f = functools.partial(
        pl.kernel,
        out_type=[
            jax.ShapeDtypeStruct((2 * NPAD,), jnp.float32),
            jax.ShapeDtypeStruct((2 * NPAD,), jnp.float32),
            jax.ShapeDtypeStruct((ECH, CH), jnp.float32),
        ],
        mesh=_SC_MESH,
        scratch_types=[
            pltpu.VMEM((NPAD,), jnp.float32),
            pltpu.VMEM((NPAD,), jnp.float32),
            pltpu.VMEM((NCH, CH), jnp.int32),
            pltpu.VMEM((NCH, CH), jnp.int32),
            pltpu.VMEM((NCH, CH), jnp.int32),
            pltpu.VMEM((NCH, CH), jnp.float32),
            pltpu.VMEM((CH,), jnp.float32),
            pltpu.VMEM((TROW,), jnp.float32),
            pltpu.VMEM((16,), jnp.float32),
            pltpu.VMEM_SHARED((NPAD,), jnp.float32),
            pltpu.VMEM_SHARED((NPAD,), jnp.float32),
        ],
        compiler_params=pltpu.CompilerParams(needs_layout_passes=False),
    )(_sc1_body)
    return f(td2, fs2, fd2, asrc, adst, m16)


WIN = 16
NWIN = NCH // WIN
NWIN2 = NCH2 // WIN


def _zero_acc_rows(rows, acc_sh, sid):
    def _zbody(i, _):
        for c in range(D // 16):
            rows[i, pl.ds(c * 16, 16)] = jnp.zeros((16,), jnp.float32)
        return _
    lax.fori_loop(0, CH, _zbody, None)
    for k in range(TROW // CH):
        pltpu.sync_copy(rows.at[pl.ds(0, CH)],
                        acc_sh.at[pl.ds(sid * TROW + k * CH, CH)])


def _sc2_body(hs_hbm, s2_hbm, d2_hbm, out_hbm,
              sidx, didx, rows, acc_sh, sem0, sem1):
    cid = lax.axis_index("c")
    sid = lax.axis_index("s")
    wid = cid * NSUB + sid

    _zero_acc_rows(rows, acc_sh, sid)
    plsc.subcore_barrier()

    sems = (sem0, sem1)

    def _start(j, b):
        pltpu.async_copy(hs_hbm.at[sidx.at[j]],
                         rows.at[pl.ds(b * CH, CH)], sems[b])

    def _wait(j, b):
        pltpu.make_async_copy(hs_hbm.at[sidx.at[j]],
                              rows.at[pl.ds(b * CH, CH)], sems[b]).wait()

    def _window(w, _):
        wrow = wid * NCH + w * WIN
        pltpu.sync_copy(s2_hbm.at[pl.ds(wrow, WIN)], sidx)
        pltpu.sync_copy(d2_hbm.at[pl.ds(wrow, WIN)], didx)
        _start(0, 0)
        _start(1, 1)

        def _loop(j0, __):
            for b in range(2):
                j = 2 * j0 + b
                _wait(j, b)
                pltpu.sync_copy(rows.at[pl.ds(b * CH, CH)],
                                acc_sh.at[didx.at[j]], add=True)

                @pl.when(j + 2 < WIN)
                def _():
                    _start(j + 2, b)
            return __
        lax.fori_loop(0, WIN // 2, _loop, None)
        return _
    lax.fori_loop(0, NWIN, _window, None)

    plsc.subcore_barrier()
    base = cid * NPAD + sid * TROW
    pltpu.sync_copy(acc_sh.at[pl.ds(sid * TROW, TROW)],
                    out_hbm.at[pl.ds(base, TROW)])


def _sc_stage2(hs1, ts2, td2):
    f = functools.partial(
        pl.kernel,
        out_type=jax.ShapeDtypeStruct((2 * NPAD, D), jnp.float32),
        mesh=_SC_MESH,
        scratch_types=[
            pltpu.VMEM((WIN, CH), jnp.int32),
            pltpu.VMEM((WIN, CH), jnp.int32),
            pltpu.VMEM((2 * CH, D), jnp.float32),
            pltpu.VMEM_SHARED((NPAD, D), jnp.float32),
            pltpu.SemaphoreType.DMA,
            pltpu.SemaphoreType.DMA,
        ],
        compiler_params=pltpu.CompilerParams(needs_layout_passes=False),
    )(_sc2_body)
    return f(hs1, ts2, td2)


def _sc_gat_body(hg_hbm, fs2_hbm, fd2_hbm, ex_hbm, out_hbm,
                 sidx, didx, exv, rows, acc_sh, sem0, sem1):
    cid = lax.axis_index("c")
    sid = lax.axis_index("s")
    wid = cid * NSUB + sid

    _zero_acc_rows(rows, acc_sh, sid)
    plsc.subcore_barrier()
    sems = (sem0, sem1)

    def _start(j, b):
        pltpu.async_copy(hg_hbm.at[sidx.at[j]],
                         rows.at[pl.ds(b * CH, CH)], sems[b])

    def _wait(j, b):
        pltpu.make_async_copy(hg_hbm.at[sidx.at[j]],
                              rows.at[pl.ds(b * CH, CH)], sems[b]).wait()

    def _window(w, _):
        wrow = wid * NCH + w * WIN
        pltpu.sync_copy(fs2_hbm.at[pl.ds(wrow, WIN)], sidx)
        pltpu.sync_copy(fd2_hbm.at[pl.ds(wrow, WIN)], didx)
        pltpu.sync_copy(ex_hbm.at[pl.ds(wrow, WIN)], exv)
        _start(0, 0)
        _start(1, 1)

        def _loop(j0, __):
            for b in range(2):
                j = 2 * j0 + b
                _wait(j, b)
                for g in range(CH // 16):
                    ev = exv[j, pl.ds(g * 16, 16)]
                    for i in range(16):
                        r = g * 16 + i
                        sc = ev[i]
                        for c in range(D // 16):
                            sl = pl.ds(c * 16, 16)
                            rows[b * CH + r, sl] = rows[b * CH + r, sl] * sc
                pltpu.sync_copy(rows.at[pl.ds(b * CH, CH)],
                                acc_sh.at[didx.at[j]], add=True)

                @pl.when(j + 2 < WIN)
                def _():
                    _start(j + 2, b)
            return __
        lax.fori_loop(0, WIN // 2, _loop, None)
        return _
    lax.fori_loop(0, NWIN, _window, None)

    plsc.subcore_barrier()
    base = cid * NPAD + sid * TROW
    pltpu.sync_copy(acc_sh.at[pl.ds(sid * TROW, TROW)],
                    out_hbm.at[pl.ds(base, TROW)])


def _sc_gat(hg, fs2, fd2, ex2):
    f = functools.partial(
        pl.kernel,
        out_type=jax.ShapeDtypeStruct((2 * NPAD, D), jnp.float32),
        mesh=_SC_MESH,
        scratch_types=[
            pltpu.VMEM((WIN, CH), jnp.int32),
            pltpu.VMEM((WIN, CH), jnp.int32),
            pltpu.VMEM((WIN, CH), jnp.float32),
            pltpu.VMEM((2 * CH, D), jnp.float32),
            pltpu.VMEM_SHARED((NPAD, D), jnp.float32),
            pltpu.SemaphoreType.DMA,
            pltpu.SemaphoreType.DMA,
        ],
        compiler_params=pltpu.CompilerParams(needs_layout_passes=False),
    )(_sc_gat_body)
    return f(hg, fs2, fd2, ex2)


def kernel(topo_x, topo_edge_index, feat_x, feat_edge_index,
           W1, b1, W2, b2, Wg, att_l, att_r, Wd):
    pad_src = jnp.zeros((E2 - E,), jnp.int32)
    pad_dst = jnp.full((E2 - E,), N, jnp.int32)
    ts2 = jnp.concatenate([topo_edge_index[0], pad_src]).reshape(ECH, CH)
    td2 = jnp.concatenate([topo_edge_index[1], pad_dst]).reshape(ECH, CH)
    fs2 = jnp.concatenate([feat_edge_index[0], pad_src]).reshape(ECH, CH)
    fd2 = jnp.concatenate([feat_edge_index[1], pad_dst]).reshape(ECH, CH)

    h1, hg, asrc, adst, mm = _tc_stage1(topo_x, feat_x, W1, Wg, att_l, att_r)

    m16 = jnp.zeros((16,), jnp.float32).at[:2].set(mm.reshape(2))
    padn = jnp.zeros((NPAD - N,), jnp.float32)
    asrc_p = jnp.concatenate([asrc.reshape(N), padn])
    adst_p = jnp.concatenate([adst.reshape(N), padn])
    deg_flat, s_flat, ex2 = _sc_stage1(td2, fs2, fd2, asrc_p, adst_p, m16)

    deg0 = deg_flat[:N].reshape(N, 1)
    deg1 = deg_flat[NPAD:NPAD + N].reshape(N, 1)
    dinv, hs1 = _tc_stage2(deg0, deg1, h1)

    acc1_flat = _sc_stage2(hs1, ts2, td2)
    acc1a = acc1_flat[:N]
    acc1b = acc1_flat[NPAD:NPAD + N]

    h2, hs2 = _tc_stage3(acc1a, acc1b, h1, dinv, b1, W2)

    acc2_flat = _sc_stage2(hs2, ts2, td2)
    accg_flat = _sc_gat(hg, fs2, fd2, ex2)

    s0 = s_flat[:N].reshape(N, 1)
    s1 = s_flat[NPAD:NPAD + N].reshape(N, 1)
    return _tc_stage4(acc2_flat[:N], acc2_flat[NPAD:NPAD + N],
                      accg_flat[:N], accg_flat[NPAD:NPAD + N],
                      h2, hg, dinv, s0, s1, asrc, adst, mm, b2, Wd)


# speedup vs baseline: 1.3748x; 1.3748x over previous
import functools

import jax
import jax.numpy as jnp
from jax import lax
from jax.experimental import pallas as pl
from jax.experimental.pallas import tpu as pltpu
from jax.experimental.pallas import tpu_sc as plsc

N = 10000
D = 128
BLK = 1000
GRID = N // BLK
NEG_INF = -3.0e38

E = 320000
NW = 32
NSUB = 16
CH = 80
E2 = 327680
ECH = E2 // CH
NCH = E2 // NW // CH
NCH2 = E2 // NSUB // CH
NPAD = 10240
TROW = NPAD // NSUB


def _lrelu(x):
    return jnp.where(x > 0, x, 0.2 * x)


def _tc1_body(xt_ref, xf_ref, w1_ref, wg_ref, al_ref, ar_ref,
              h1_ref, hg_ref, asrc_ref, adst_ref, m_ref):
    i = pl.program_id(0)
    h1 = jnp.dot(xt_ref[...], w1_ref[...], preferred_element_type=jnp.float32)
    h1_ref[...] = h1
    hg = jnp.dot(xf_ref[...], wg_ref[...], preferred_element_type=jnp.float32)
    hg_ref[...] = hg
    asrc = jnp.sum(hg * al_ref[...], axis=-1)
    adst = jnp.sum(hg * ar_ref[...], axis=-1)
    asrc_ref[...] = asrc[:, None]
    adst_ref[...] = adst[:, None]

    cur = jnp.where(i == 0, jnp.full((1, 2), NEG_INF, jnp.float32), m_ref[...])
    new = jnp.stack([jnp.max(asrc), jnp.max(adst)]).reshape(1, 2)
    m_ref[...] = jnp.maximum(cur, new)


def _tc_stage1(topo_x, feat_x, W1, Wg, att_l, att_r):
    return pl.pallas_call(
        _tc1_body,
        grid=(GRID,),
        in_specs=[
            pl.BlockSpec((BLK, D), lambda i: (i, 0)),
            pl.BlockSpec((BLK, D), lambda i: (i, 0)),
            pl.BlockSpec((D, D), lambda i: (0, 0)),
            pl.BlockSpec((D, D), lambda i: (0, 0)),
            pl.BlockSpec((1, D), lambda i: (0, 0)),
            pl.BlockSpec((1, D), lambda i: (0, 0)),
        ],
        out_specs=[
            pl.BlockSpec((BLK, D), lambda i: (i, 0)),
            pl.BlockSpec((BLK, D), lambda i: (i, 0)),
            pl.BlockSpec((BLK, 1), lambda i: (i, 0)),
            pl.BlockSpec((BLK, 1), lambda i: (i, 0)),
            pl.BlockSpec((1, 2), lambda i: (0, 0)),
        ],
        out_shape=[
            jax.ShapeDtypeStruct((N, D), jnp.float32),
            jax.ShapeDtypeStruct((N, D), jnp.float32),
            jax.ShapeDtypeStruct((N, 1), jnp.float32),
            jax.ShapeDtypeStruct((N, 1), jnp.float32),
            jax.ShapeDtypeStruct((1, 2), jnp.float32),
        ],
    )(topo_x, feat_x, W1, Wg, att_l.reshape(1, D), att_r.reshape(1, D))


def _tc2_body(d0_ref, d1_ref, h1_ref, dinv_ref, hs1_ref):
    deg = d0_ref[:, 0] + d1_ref[:, 0] + 1.0
    dinv = lax.rsqrt(deg)
    dinv_ref[...] = dinv[:, None]
    hs1_ref[...] = h1_ref[...] * dinv[:, None]


def _tc_stage2(deg0, deg1, h1):
    return pl.pallas_call(
        _tc2_body,
        grid=(GRID,),
        in_specs=[
            pl.BlockSpec((BLK, 1), lambda i: (i, 0)),
            pl.BlockSpec((BLK, 1), lambda i: (i, 0)),
            pl.BlockSpec((BLK, D), lambda i: (i, 0)),
        ],
        out_specs=[
            pl.BlockSpec((BLK, 1), lambda i: (i, 0)),
            pl.BlockSpec((BLK, D), lambda i: (i, 0)),
        ],
        out_shape=[
            jax.ShapeDtypeStruct((N, 1), jnp.float32),
            jax.ShapeDtypeStruct((N, D), jnp.float32),
        ],
    )(deg0, deg1, h1)


def _tc3_body(a0_ref, a1_ref, h1_ref, dinv_ref, b1_ref, w2_ref,
              h2_ref, hs2_ref):
    di = dinv_ref[:, 0]
    a = a0_ref[...] + a1_ref[...]
    x2 = jnp.maximum(
        di[:, None] * a + (di * di)[:, None] * h1_ref[...] + b1_ref[...], 0.0)
    h2 = jnp.dot(x2, w2_ref[...], preferred_element_type=jnp.float32)
    h2_ref[...] = h2
    hs2_ref[...] = h2 * di[:, None]


def _tc_stage3(acc1a, acc1b, h1, dinv, b1, W2):
    return pl.pallas_call(
        _tc3_body,
        grid=(GRID,),
        in_specs=[
            pl.BlockSpec((BLK, D), lambda i: (i, 0)),
            pl.BlockSpec((BLK, D), lambda i: (i, 0)),
            pl.BlockSpec((BLK, D), lambda i: (i, 0)),
            pl.BlockSpec((BLK, 1), lambda i: (i, 0)),
            pl.BlockSpec((1, D), lambda i: (0, 0)),
            pl.BlockSpec((D, D), lambda i: (0, 0)),
        ],
        out_specs=[
            pl.BlockSpec((BLK, D), lambda i: (i, 0)),
            pl.BlockSpec((BLK, D), lambda i: (i, 0)),
        ],
        out_shape=[
            jax.ShapeDtypeStruct((N, D), jnp.float32),
            jax.ShapeDtypeStruct((N, D), jnp.float32),
        ],
    )(acc1a, acc1b, h1, dinv, b1.reshape(1, D), W2)


def _tc4_body(a2_ref, ag_ref, h2_ref, hg_ref, dinv_ref,
              s0_ref, s1_ref, asrc_ref, adst_ref, m_ref, b2_ref, wd_ref,
              res_ref):
    di = dinv_ref[:, 0]
    topo_z = (di[:, None] * a2_ref[...] + (di * di)[:, None] * h2_ref[...]
              + b2_ref[...])
    mv = m_ref[...]
    M = _lrelu(mv[0, 0] + mv[0, 1])
    ex_self = jnp.exp(_lrelu(asrc_ref[:, 0] + adst_ref[:, 0]) - M)
    s = s0_ref[:, 0] + s1_ref[:, 0] + ex_self
    feat_z = (ag_ref[...] + ex_self[:, None] * hg_ref[...]) / (
        s[:, None] + 1e-16)
    fzw = jnp.dot(feat_z, wd_ref[...], preferred_element_type=jnp.float32)
    res = jax.nn.sigmoid(jnp.sum(topo_z * fzw, axis=-1))
    res_ref[...] = res[:, None]


def _tc_stage4(a2, ag, h2, hg, dinv, s0, s1, asrc, adst, mm, b2, Wd):
    res = pl.pallas_call(
        _tc4_body,
        grid=(GRID,),
        in_specs=[
            pl.BlockSpec((BLK, D), lambda i: (i, 0)),
            pl.BlockSpec((BLK, D), lambda i: (i, 0)),
            pl.BlockSpec((BLK, D), lambda i: (i, 0)),
            pl.BlockSpec((BLK, D), lambda i: (i, 0)),
            pl.BlockSpec((BLK, 1), lambda i: (i, 0)),
            pl.BlockSpec((BLK, 1), lambda i: (i, 0)),
            pl.BlockSpec((BLK, 1), lambda i: (i, 0)),
            pl.BlockSpec((BLK, 1), lambda i: (i, 0)),
            pl.BlockSpec((BLK, 1), lambda i: (i, 0)),
            pl.BlockSpec((1, 2), lambda i: (0, 0)),
            pl.BlockSpec((1, D), lambda i: (0, 0)),
            pl.BlockSpec((D, D), lambda i: (0, 0)),
        ],
        out_specs=[pl.BlockSpec((BLK, 1), lambda i: (i, 0))],
        out_shape=[jax.ShapeDtypeStruct((N, 1), jnp.float32)],
    )(a2, ag, h2, hg, dinv, s0, s1, asrc, adst, mm,
      b2.reshape(1, D), Wd)[0]
    return res.reshape(N)


_SC_MESH = plsc.VectorSubcoreMesh(core_axis_name="c", subcore_axis_name="s")


def _sc1_body(td_hbm, fs_hbm, fd_hbm, asrc_hbm, adst_hbm, m_hbm,
              deg_out, s_out, ex_out,
              asrc_v, adst_v, tdi, fsi, fdi, exv, ones_v, zv, m_v,
              deg_sh, s_sh):
    cid = lax.axis_index("c")
    sid = lax.axis_index("s")
    wid = cid * NSUB + sid

    def _zbody(i, _):
        zv[pl.ds(i * 16, 16)] = jnp.zeros((16,), jnp.float32)
        return _
    lax.fori_loop(0, TROW // 16, _zbody, None)
    pltpu.sync_copy(zv, deg_sh.at[pl.ds(sid * TROW, TROW)])
    pltpu.sync_copy(zv, s_sh.at[pl.ds(sid * TROW, TROW)])

    def _obody(i, _):
        ones_v[pl.ds(i * 16, 16)] = jnp.ones((16,), jnp.float32)
        return _
    lax.fori_loop(0, CH // 16, _obody, None)

    pltpu.sync_copy(asrc_hbm, asrc_v)
    pltpu.sync_copy(adst_hbm, adst_v)
    pltpu.sync_copy(m_hbm, m_v)
    pltpu.sync_copy(td_hbm.at[pl.ds(wid * NCH, NCH)], tdi)
    pltpu.sync_copy(fs_hbm.at[pl.ds(wid * NCH, NCH)], fsi)
    pltpu.sync_copy(fd_hbm.at[pl.ds(wid * NCH, NCH)], fdi)
    plsc.subcore_barrier()

    mv = m_v[pl.ds(0, 16)]
    M = _lrelu(mv[0] + mv[1])

    def _chunk(j, _):
        for r in range(CH // 16):
            sv = fsi[j, pl.ds(r * 16, 16)]
            dv = fdi[j, pl.ds(r * 16, 16)]
            av = plsc.load_gather(asrc_v, [sv])
            bv = plsc.load_gather(adst_v, [dv])
            ex = jnp.exp(_lrelu(av + bv) - M)
            exv[j, pl.ds(r * 16, 16)] = ex
        pltpu.sync_copy(exv.at[j], s_sh.at[fdi.at[j]], add=True)
        pltpu.sync_copy(ones_v, deg_sh.at[tdi.at[j]], add=True)
        return _
    lax.fori_loop(0, NCH, _chunk, None)

    pltpu.sync_copy(exv, ex_out.at[pl.ds(wid * NCH, NCH)])
    plsc.subcore_barrier()

    base = cid * NPAD + sid * TROW
    pltpu.sync_copy(deg_sh.at[pl.ds(sid * TROW, TROW)],
                    deg_out.at[pl.ds(base, TROW)])
    pltpu.sync_copy(s_sh.at[pl.ds(sid * TROW, TROW)],
                    s_out.at[pl.ds(base, TROW)])


def _sc_stage1(td2, fs2, fd2, asrc, adst, m16):
    f = functools.partial(
        pl.kernel,
        out_type=[
            jax.ShapeDtypeStruct((2 * NPAD,), jnp.float32),
            jax.ShapeDtypeStruct((2 * NPAD,), jnp.float32),
            jax.ShapeDtypeStruct((ECH, CH), jnp.float32),
        ],
        mesh=_SC_MESH,
        scratch_types=[
            pltpu.VMEM((NPAD,), jnp.float32),
            pltpu.VMEM((NPAD,), jnp.float32),
            pltpu.VMEM((NCH, CH), jnp.int32),
            pltpu.VMEM((NCH, CH), jnp.int32),
            pltpu.VMEM((NCH, CH), jnp.int32),
            pltpu.VMEM((NCH, CH), jnp.float32),
            pltpu.VMEM((CH,), jnp.float32),
            pltpu.VMEM((TROW,), jnp.float32),
            pltpu.VMEM((16,), jnp.float32),
            pltpu.VMEM_SHARED((NPAD,), jnp.float32),
            pltpu.VMEM_SHARED((NPAD,), jnp.float32),
        ],
        compiler_params=pltpu.CompilerParams(needs_layout_passes=False),
    )(_sc1_body)
    return f(td2, fs2, fd2, asrc, adst, m16)


CHR = 40
ECHR = E2 // CHR
NCHR = E2 // NW // CHR
NCHR2 = E2 // NSUB // CHR
RWIN = 64
NWINR = NCHR // RWIN
NWINR2 = NCHR2 // RWIN


def _zero_acc_rows(rows, acc_sh, sid):
    def _zbody(i, _):
        for c in range(D // 16):
            rows[i, pl.ds(c * 16, 16)] = jnp.zeros((16,), jnp.float32)
        return _
    lax.fori_loop(0, CHR, _zbody, None)
    for k in range(TROW // CHR):
        pltpu.sync_copy(rows.at[pl.ds(0, CHR)],
                        acc_sh.at[pl.ds(sid * TROW + k * CHR, CHR)])


def _scale_rows(rows, exv, j, base):
    for off, lo in ((0, 0), (16, 0), (24, 8)):
        ev = exv[j, pl.ds(off, 16)]
        for i in range(lo, 16):
            r = off + i
            sc = ev[i]
            for c in range(D // 16):
                sl = pl.ds(c * 16, 16)
                rows[base + r, sl] = rows[base + r, sl] * sc


def _ring_pass(table, srcc, dstc, exs, sidx, didx, exv, rows, acc_sh,
               gsems, ssems, nwin, chunk_base, scale):
    def _startg(j, b):
        pltpu.async_copy(table.at[sidx.at[j]],
                         rows.at[pl.ds(b * CHR, CHR)], gsems[b])

    def _waitg(j, b):
        pltpu.make_async_copy(table.at[sidx.at[j]],
                              rows.at[pl.ds(b * CHR, CHR)], gsems[b]).wait()

    def _starts(j, b):
        pltpu.async_copy(rows.at[pl.ds(b * CHR, CHR)],
                         acc_sh.at[didx.at[j]], ssems[b], add=True)

    def _waits(j, b):
        pltpu.make_async_copy(rows.at[pl.ds(b * CHR, CHR)],
                              acc_sh.at[didx.at[j]], ssems[b]).wait()

    def _window(w, _):
        wrow = chunk_base + w * RWIN
        pltpu.sync_copy(srcc.at[pl.ds(wrow, RWIN)], sidx)
        pltpu.sync_copy(dstc.at[pl.ds(wrow, RWIN)], didx)
        if scale:
            pltpu.sync_copy(exs.at[pl.ds(wrow, RWIN)], exv)
        _startg(0, 0)
        _startg(1, 1)

        def _ring(j0, __):
            for b in range(4):
                j = 4 * j0 + b
                _waitg(j, b)
                if scale:
                    _scale_rows(rows, exv, j, b * CHR)
                _starts(j, b)

                @pl.when(j >= 2)
                def _():
                    _waits(j - 2, (b - 2) % 4)

                @pl.when(j + 2 < RWIN)
                def _():
                    _startg(j + 2, (b + 2) % 4)
            return __
        lax.fori_loop(0, RWIN // 4, _ring, None)
        _waits(RWIN - 2, (RWIN - 2) % 4)
        _waits(RWIN - 1, (RWIN - 1) % 4)
        return _
    lax.fori_loop(0, nwin, _window, None)


def _sc2_body(hs_hbm, sc_hbm, dc_hbm, out_hbm,
              sidx, didx, rows, acc_sh,
              g0, g1, g2, g3, ss0, ss1, ss2, ss3):
    cid = lax.axis_index("c")
    sid = lax.axis_index("s")
    wid = cid * NSUB + sid

    _zero_acc_rows(rows, acc_sh, sid)
    plsc.subcore_barrier()
    _ring_pass(hs_hbm, sc_hbm, dc_hbm, None, sidx, didx, None, rows, acc_sh,
               (g0, g1, g2, g3), (ss0, ss1, ss2, ss3),
               NWINR, wid * NCHR, False)
    plsc.subcore_barrier()
    base = cid * NPAD + sid * TROW
    pltpu.sync_copy(acc_sh.at[pl.ds(sid * TROW, TROW)],
                    out_hbm.at[pl.ds(base, TROW)])


def _sc_stage2(hs1, ts2r, td2r):
    f = functools.partial(
        pl.kernel,
        out_type=jax.ShapeDtypeStruct((2 * NPAD, D), jnp.float32),
        mesh=_SC_MESH,
        scratch_types=[
            pltpu.VMEM((RWIN, CHR), jnp.int32),
            pltpu.VMEM((RWIN, CHR), jnp.int32),
            pltpu.VMEM((4 * CHR, D), jnp.float32),
            pltpu.VMEM_SHARED((NPAD, D), jnp.float32),
        ] + [pltpu.SemaphoreType.DMA] * 8,
        compiler_params=pltpu.CompilerParams(needs_layout_passes=False),
    )(_sc2_body)
    return f(hs1, ts2r, td2r)


def _sc3_body(hs2_hbm, hg_hbm, tsc_hbm, tdc_hbm, fsc_hbm, fdc_hbm, ex_hbm,
              acc2_out, accg_out,
              sidx, didx, exv, rows, acc_sh,
              g0, g1, g2, g3, ss0, ss1, ss2, ss3):
    cid = lax.axis_index("c")
    sid = lax.axis_index("s")
    gsems = (g0, g1, g2, g3)
    ssems = (ss0, ss1, ss2, ss3)

    _zero_acc_rows(rows, acc_sh, sid)
    plsc.subcore_barrier()

    @pl.when(cid == 0)
    def _():
        _ring_pass(hs2_hbm, tsc_hbm, tdc_hbm, None, sidx, didx, None, rows,
                   acc_sh, gsems, ssems, NWINR2, sid * NCHR2, False)

    @pl.when(cid == 1)
    def _():
        _ring_pass(hg_hbm, fsc_hbm, fdc_hbm, ex_hbm, sidx, didx, exv, rows,
                   acc_sh, gsems, ssems, NWINR2, sid * NCHR2, True)

    plsc.subcore_barrier()
    base = sid * TROW

    @pl.when(cid == 0)
    def _():
        pltpu.sync_copy(acc_sh.at[pl.ds(base, TROW)],
                        acc2_out.at[pl.ds(base, TROW)])

    @pl.when(cid == 1)
    def _():
        pltpu.sync_copy(acc_sh.at[pl.ds(base, TROW)],
                        accg_out.at[pl.ds(base, TROW)])


def _sc_stage3(hs2, hg, ts2r, td2r, fs2r, fd2r, ex2r):
    f = functools.partial(
        pl.kernel,
        out_type=[
            jax.ShapeDtypeStruct((NPAD, D), jnp.float32),
            jax.ShapeDtypeStruct((NPAD, D), jnp.float32),
        ],
        mesh=_SC_MESH,
        scratch_types=[
            pltpu.VMEM((RWIN, CHR), jnp.int32),
            pltpu.VMEM((RWIN, CHR), jnp.int32),
            pltpu.VMEM((RWIN, CHR), jnp.float32),
            pltpu.VMEM((4 * CHR, D), jnp.float32),
            pltpu.VMEM_SHARED((NPAD, D), jnp.float32),
        ] + [pltpu.SemaphoreType.DMA] * 8,
        compiler_params=pltpu.CompilerParams(needs_layout_passes=False),
    )(_sc3_body)
    return f(hs2, hg, ts2r, td2r, fs2r, fd2r, ex2r)


def kernel(topo_x, topo_edge_index, feat_x, feat_edge_index,
           W1, b1, W2, b2, Wg, att_l, att_r, Wd):
    pad_src = jnp.zeros((E2 - E,), jnp.int32)
    pad_dst = jnp.full((E2 - E,), N, jnp.int32)
    ts_p = jnp.concatenate([topo_edge_index[0], pad_src])
    td_p = jnp.concatenate([topo_edge_index[1], pad_dst])
    fs_p = jnp.concatenate([feat_edge_index[0], pad_src])
    fd_p = jnp.concatenate([feat_edge_index[1], pad_dst])
    td2 = td_p.reshape(ECH, CH)
    fs2 = fs_p.reshape(ECH, CH)
    fd2 = fd_p.reshape(ECH, CH)
    ts2r = ts_p.reshape(ECHR, CHR)
    td2r = td_p.reshape(ECHR, CHR)
    fs2r = fs_p.reshape(ECHR, CHR)
    fd2r = fd_p.reshape(ECHR, CHR)

    h1, hg, asrc, adst, mm = _tc_stage1(topo_x, feat_x, W1, Wg, att_l, att_r)

    m16 = jnp.zeros((16,), jnp.float32).at[:2].set(mm.reshape(2))
    padn = jnp.zeros((NPAD - N,), jnp.float32)
    asrc_p = jnp.concatenate([asrc.reshape(N), padn])
    adst_p = jnp.concatenate([adst.reshape(N), padn])
    deg_flat, s_flat, ex2 = _sc_stage1(td2, fs2, fd2, asrc_p, adst_p, m16)

    deg0 = deg_flat[:N].reshape(N, 1)
    deg1 = deg_flat[NPAD:NPAD + N].reshape(N, 1)
    dinv, hs1 = _tc_stage2(deg0, deg1, h1)

    acc1_flat = _sc_stage2(hs1, ts2r, td2r)
    acc1a = acc1_flat[:N]
    acc1b = acc1_flat[NPAD:NPAD + N]

    h2, hs2 = _tc_stage3(acc1a, acc1b, h1, dinv, b1, W2)

    ex2r = ex2.reshape(ECHR, CHR)
    acc2, accg = _sc_stage3(hs2, hg, ts2r, td2r, fs2r, fd2r, ex2r)

    s0 = s_flat[:N].reshape(N, 1)
    s1 = s_flat[NPAD:NPAD + N].reshape(N, 1)
    return _tc_stage4(acc2[:N], accg[:N],
                      h2, hg, dinv, s0, s1, asrc, adst, mm, b2, Wd)


# speedup vs baseline: 1.3794x; 1.0033x over previous
import functools

import jax
import jax.numpy as jnp
from jax import lax
from jax.experimental import pallas as pl
from jax.experimental.pallas import tpu as pltpu
from jax.experimental.pallas import tpu_sc as plsc

N = 10000
D = 128
BLK = 1000
GRID = N // BLK
NEG_INF = -3.0e38

E = 320000
NW = 32
NSUB = 16
CH = 80
E2 = 327680
ECH = E2 // CH
NCH = E2 // NW // CH
NCH2 = E2 // NSUB // CH
NPAD = 10240
TROW = NPAD // NSUB


def _lrelu(x):
    return jnp.where(x > 0, x, 0.2 * x)


def _tc1_body(xt_ref, xf_ref, w1_ref, wg_ref, al_ref, ar_ref,
              h1_ref, hg_ref, asrc_ref, adst_ref, m_ref):
    i = pl.program_id(0)
    h1 = jnp.dot(xt_ref[...], w1_ref[...], preferred_element_type=jnp.float32)
    h1_ref[...] = h1
    hg = jnp.dot(xf_ref[...], wg_ref[...], preferred_element_type=jnp.float32)
    hg_ref[...] = hg
    asrc = jnp.sum(hg * al_ref[...], axis=-1)
    adst = jnp.sum(hg * ar_ref[...], axis=-1)
    asrc_ref[...] = asrc[:, None]
    adst_ref[...] = adst[:, None]

    cur = jnp.where(i == 0, jnp.full((1, 2), NEG_INF, jnp.float32), m_ref[...])
    new = jnp.stack([jnp.max(asrc), jnp.max(adst)]).reshape(1, 2)
    m_ref[...] = jnp.maximum(cur, new)


def _tc_stage1(topo_x, feat_x, W1, Wg, att_l, att_r):
    return pl.pallas_call(
        _tc1_body,
        grid=(GRID,),
        in_specs=[
            pl.BlockSpec((BLK, D), lambda i: (i, 0)),
            pl.BlockSpec((BLK, D), lambda i: (i, 0)),
            pl.BlockSpec((D, D), lambda i: (0, 0)),
            pl.BlockSpec((D, D), lambda i: (0, 0)),
            pl.BlockSpec((1, D), lambda i: (0, 0)),
            pl.BlockSpec((1, D), lambda i: (0, 0)),
        ],
        out_specs=[
            pl.BlockSpec((BLK, D), lambda i: (i, 0)),
            pl.BlockSpec((BLK, D), lambda i: (i, 0)),
            pl.BlockSpec((BLK, 1), lambda i: (i, 0)),
            pl.BlockSpec((BLK, 1), lambda i: (i, 0)),
            pl.BlockSpec((1, 2), lambda i: (0, 0)),
        ],
        out_shape=[
            jax.ShapeDtypeStruct((N, D), jnp.float32),
            jax.ShapeDtypeStruct((N, D), jnp.float32),
            jax.ShapeDtypeStruct((N, 1), jnp.float32),
            jax.ShapeDtypeStruct((N, 1), jnp.float32),
            jax.ShapeDtypeStruct((1, 2), jnp.float32),
        ],
    )(topo_x, feat_x, W1, Wg, att_l.reshape(1, D), att_r.reshape(1, D))


def _tc2_body(d0_ref, d1_ref, h1_ref, dinv_ref, hs1_ref):
    deg = d0_ref[:, 0] + d1_ref[:, 0] + 1.0
    dinv = lax.rsqrt(deg)
    dinv_ref[...] = dinv[:, None]
    hs1_ref[...] = h1_ref[...] * dinv[:, None]


def _tc_stage2(deg0, deg1, h1):
    return pl.pallas_call(
        _tc2_body,
        grid=(GRID,),
        in_specs=[
            pl.BlockSpec((BLK, 1), lambda i: (i, 0)),
            pl.BlockSpec((BLK, 1), lambda i: (i, 0)),
            pl.BlockSpec((BLK, D), lambda i: (i, 0)),
        ],
        out_specs=[
            pl.BlockSpec((BLK, 1), lambda i: (i, 0)),
            pl.BlockSpec((BLK, D), lambda i: (i, 0)),
        ],
        out_shape=[
            jax.ShapeDtypeStruct((N, 1), jnp.float32),
            jax.ShapeDtypeStruct((N, D), jnp.float32),
        ],
    )(deg0, deg1, h1)


def _tc3_body(a0_ref, a1_ref, h1_ref, dinv_ref, b1_ref, w2_ref,
              h2_ref, hs2_ref):
    di = dinv_ref[:, 0]
    a = a0_ref[...] + a1_ref[...]
    x2 = jnp.maximum(
        di[:, None] * a + (di * di)[:, None] * h1_ref[...] + b1_ref[...], 0.0)
    h2 = jnp.dot(x2, w2_ref[...], preferred_element_type=jnp.float32)
    h2_ref[...] = h2
    hs2_ref[...] = h2 * di[:, None]


def _tc_stage3(acc1a, acc1b, h1, dinv, b1, W2):
    return pl.pallas_call(
        _tc3_body,
        grid=(GRID,),
        in_specs=[
            pl.BlockSpec((BLK, D), lambda i: (i, 0)),
            pl.BlockSpec((BLK, D), lambda i: (i, 0)),
            pl.BlockSpec((BLK, D), lambda i: (i, 0)),
            pl.BlockSpec((BLK, 1), lambda i: (i, 0)),
            pl.BlockSpec((1, D), lambda i: (0, 0)),
            pl.BlockSpec((D, D), lambda i: (0, 0)),
        ],
        out_specs=[
            pl.BlockSpec((BLK, D), lambda i: (i, 0)),
            pl.BlockSpec((BLK, D), lambda i: (i, 0)),
        ],
        out_shape=[
            jax.ShapeDtypeStruct((N, D), jnp.float32),
            jax.ShapeDtypeStruct((N, D), jnp.float32),
        ],
    )(acc1a, acc1b, h1, dinv, b1.reshape(1, D), W2)


def _tc4_body(a2_ref, ag_ref, h2_ref, hg_ref, dinv_ref,
              s0_ref, s1_ref, asrc_ref, adst_ref, m_ref, b2_ref, wd_ref,
              res_ref):
    di = dinv_ref[:, 0]
    topo_z = (di[:, None] * a2_ref[...] + (di * di)[:, None] * h2_ref[...]
              + b2_ref[...])
    mv = m_ref[...]
    M = _lrelu(mv[0, 0] + mv[0, 1])
    ex_self = jnp.exp(_lrelu(asrc_ref[:, 0] + adst_ref[:, 0]) - M)
    s = s0_ref[:, 0] + s1_ref[:, 0] + ex_self
    feat_z = (ag_ref[...] + ex_self[:, None] * hg_ref[...]) / (
        s[:, None] + 1e-16)
    fzw = jnp.dot(feat_z, wd_ref[...], preferred_element_type=jnp.float32)
    res = jax.nn.sigmoid(jnp.sum(topo_z * fzw, axis=-1))
    res_ref[...] = res[:, None]


def _tc_stage4(a2, ag, h2, hg, dinv, s0, s1, asrc, adst, mm, b2, Wd):
    res = pl.pallas_call(
        _tc4_body,
        grid=(GRID,),
        in_specs=[
            pl.BlockSpec((BLK, D), lambda i: (i, 0)),
            pl.BlockSpec((BLK, D), lambda i: (i, 0)),
            pl.BlockSpec((BLK, D), lambda i: (i, 0)),
            pl.BlockSpec((BLK, D), lambda i: (i, 0)),
            pl.BlockSpec((BLK, 1), lambda i: (i, 0)),
            pl.BlockSpec((BLK, 1), lambda i: (i, 0)),
            pl.BlockSpec((BLK, 1), lambda i: (i, 0)),
            pl.BlockSpec((BLK, 1), lambda i: (i, 0)),
            pl.BlockSpec((BLK, 1), lambda i: (i, 0)),
            pl.BlockSpec((1, 2), lambda i: (0, 0)),
            pl.BlockSpec((1, D), lambda i: (0, 0)),
            pl.BlockSpec((D, D), lambda i: (0, 0)),
        ],
        out_specs=[pl.BlockSpec((BLK, 1), lambda i: (i, 0))],
        out_shape=[jax.ShapeDtypeStruct((N, 1), jnp.float32)],
    )(a2, ag, h2, hg, dinv, s0, s1, asrc, adst, mm,
      b2.reshape(1, D), Wd)[0]
    return res.reshape(N)


_SC_MESH = plsc.VectorSubcoreMesh(core_axis_name="c", subcore_axis_name="s")


def _sc1_body(td_hbm, fs_hbm, fd_hbm, asrc_hbm, adst_hbm, m_hbm,
              deg_out, s_out, ex_out,
              asrc_v, adst_v, tdi, fsi, fdi, exv, ones_v, zv, m_v,
              deg_sh, s_sh):
    cid = lax.axis_index("c")
    sid = lax.axis_index("s")
    wid = cid * NSUB + sid

    def _zbody(i, _):
        zv[pl.ds(i * 16, 16)] = jnp.zeros((16,), jnp.float32)
        return _
    lax.fori_loop(0, TROW // 16, _zbody, None)
    pltpu.sync_copy(zv, deg_sh.at[pl.ds(sid * TROW, TROW)])
    pltpu.sync_copy(zv, s_sh.at[pl.ds(sid * TROW, TROW)])

    def _obody(i, _):
        ones_v[pl.ds(i * 16, 16)] = jnp.ones((16,), jnp.float32)
        return _
    lax.fori_loop(0, CH // 16, _obody, None)

    pltpu.sync_copy(asrc_hbm, asrc_v)
    pltpu.sync_copy(adst_hbm, adst_v)
    pltpu.sync_copy(m_hbm, m_v)
    pltpu.sync_copy(td_hbm.at[pl.ds(wid * NCH, NCH)], tdi)
    pltpu.sync_copy(fs_hbm.at[pl.ds(wid * NCH, NCH)], fsi)
    pltpu.sync_copy(fd_hbm.at[pl.ds(wid * NCH, NCH)], fdi)
    plsc.subcore_barrier()

    mv = m_v[pl.ds(0, 16)]
    M = _lrelu(mv[0] + mv[1])

    def _chunk(j, _):
        for r in range(CH // 16):
            sv = fsi[j, pl.ds(r * 16, 16)]
            dv = fdi[j, pl.ds(r * 16, 16)]
            av = plsc.load_gather(asrc_v, [sv])
            bv = plsc.load_gather(adst_v, [dv])
            ex = jnp.exp(_lrelu(av + bv) - M)
            exv[j, pl.ds(r * 16, 16)] = ex
        pltpu.sync_copy(exv.at[j], s_sh.at[fdi.at[j]], add=True)
        pltpu.sync_copy(ones_v, deg_sh.at[tdi.at[j]], add=True)
        return _
    lax.fori_loop(0, NCH, _chunk, None)

    pltpu.sync_copy(exv, ex_out.at[pl.ds(wid * NCH, NCH)])
    plsc.subcore_barrier()

    base = cid * NPAD + sid * TROW
    pltpu.sync_copy(deg_sh.at[pl.ds(sid * TROW, TROW)],
                    deg_out.at[pl.ds(base, TROW)])
    pltpu.sync_copy(s_sh.at[pl.ds(sid * TROW, TROW)],
                    s_out.at[pl.ds(base, TROW)])


def _sc_stage1(td2, fs2, fd2, asrc, adst, m16):
    f = functools.partial(
        pl.kernel,
        out_type=[
            jax.ShapeDtypeStruct((2 * NPAD,), jnp.float32),
            jax.ShapeDtypeStruct((2 * NPAD,), jnp.float32),
            jax.ShapeDtypeStruct((ECH, CH), jnp.float32),
        ],
        mesh=_SC_MESH,
        scratch_types=[
            pltpu.VMEM((NPAD,), jnp.float32),
            pltpu.VMEM((NPAD,), jnp.float32),
            pltpu.VMEM((NCH, CH), jnp.int32),
            pltpu.VMEM((NCH, CH), jnp.int32),
            pltpu.VMEM((NCH, CH), jnp.int32),
            pltpu.VMEM((NCH, CH), jnp.float32),
            pltpu.VMEM((CH,), jnp.float32),
            pltpu.VMEM((TROW,), jnp.float32),
            pltpu.VMEM((16,), jnp.float32),
            pltpu.VMEM_SHARED((NPAD,), jnp.float32),
            pltpu.VMEM_SHARED((NPAD,), jnp.float32),
        ],
        compiler_params=pltpu.CompilerParams(needs_layout_passes=False),
    )(_sc1_body)
    return f(td2, fs2, fd2, asrc, adst, m16)


CHR = 40
ECHR = E2 // CHR
NCHR = E2 // NW // CHR
NCHR2 = E2 // NSUB // CHR
RWIN = 64
NWINR = NCHR // RWIN
NWINR2 = NCHR2 // RWIN


def _zero_acc_rows(rows, acc_sh, sid, sems):
    def _zbody(i, _):
        for c in range(D // 16):
            rows[i, pl.ds(c * 16, 16)] = jnp.zeros((16,), jnp.float32)
        return _
    lax.fori_loop(0, CHR, _zbody, None)
    nk = TROW // CHR
    for k in range(nk):
        pltpu.async_copy(rows.at[pl.ds(0, CHR)],
                         acc_sh.at[pl.ds(sid * TROW + k * CHR, CHR)],
                         sems[k % 4])
    for k in range(nk):
        pltpu.make_async_copy(rows.at[pl.ds(0, CHR)],
                              acc_sh.at[pl.ds(sid * TROW + k * CHR, CHR)],
                              sems[k % 4]).wait()


def _scale_rows(rows, exv, j, base):
    for off, lo in ((0, 0), (16, 0), (24, 8)):
        ev = exv[j, pl.ds(off, 16)]
        for i in range(lo, 16):
            r = off + i
            sc = ev[i]
            for c in range(D // 16):
                sl = pl.ds(c * 16, 16)
                rows[base + r, sl] = rows[base + r, sl] * sc


def _ring_pass(table, srcc, dstc, exs, sidx, didx, exv, rows, acc_sh,
               gsems, ssems, nwin, chunk_base, scale):
    def _startg(j, b):
        pltpu.async_copy(table.at[sidx.at[j]],
                         rows.at[pl.ds(b * CHR, CHR)], gsems[b])

    def _waitg(j, b):
        pltpu.make_async_copy(table.at[sidx.at[j]],
                              rows.at[pl.ds(b * CHR, CHR)], gsems[b]).wait()

    def _starts(j, b):
        pltpu.async_copy(rows.at[pl.ds(b * CHR, CHR)],
                         acc_sh.at[didx.at[j]], ssems[b], add=True)

    def _waits(j, b):
        pltpu.make_async_copy(rows.at[pl.ds(b * CHR, CHR)],
                              acc_sh.at[didx.at[j]], ssems[b]).wait()

    def _window(w, _):
        wrow = chunk_base + w * RWIN
        pltpu.sync_copy(srcc.at[pl.ds(wrow, RWIN)], sidx)
        pltpu.sync_copy(dstc.at[pl.ds(wrow, RWIN)], didx)
        if scale:
            pltpu.sync_copy(exs.at[pl.ds(wrow, RWIN)], exv)
        _startg(0, 0)
        _startg(1, 1)

        def _ring(j0, __):
            for b in range(4):
                j = 4 * j0 + b
                _waitg(j, b)
                if scale:
                    _scale_rows(rows, exv, j, b * CHR)
                _starts(j, b)

                @pl.when(j >= 2)
                def _():
                    _waits(j - 2, (b - 2) % 4)

                @pl.when(j + 2 < RWIN)
                def _():
                    _startg(j + 2, (b + 2) % 4)
            return __
        lax.fori_loop(0, RWIN // 4, _ring, None)
        _waits(RWIN - 2, (RWIN - 2) % 4)
        _waits(RWIN - 1, (RWIN - 1) % 4)
        return _
    lax.fori_loop(0, nwin, _window, None)


def _sc2_body(hs_hbm, sc_hbm, dc_hbm, out_hbm,
              sidx, didx, rows, acc_sh,
              g0, g1, g2, g3, ss0, ss1, ss2, ss3):
    cid = lax.axis_index("c")
    sid = lax.axis_index("s")
    wid = sid * 2 + cid

    _zero_acc_rows(rows, acc_sh, sid, (g0, g1, g2, g3))
    plsc.subcore_barrier()
    _ring_pass(hs_hbm, sc_hbm, dc_hbm, None, sidx, didx, None, rows, acc_sh,
               (g0, g1, g2, g3), (ss0, ss1, ss2, ss3),
               NWINR, wid * NCHR, False)
    plsc.subcore_barrier()
    base = cid * NPAD + sid * TROW
    pltpu.sync_copy(acc_sh.at[pl.ds(sid * TROW, TROW)],
                    out_hbm.at[pl.ds(base, TROW)])


def _sc_stage2(hs1, ts2r, td2r):
    f = functools.partial(
        pl.kernel,
        out_type=jax.ShapeDtypeStruct((2 * NPAD, D), jnp.float32),
        mesh=_SC_MESH,
        scratch_types=[
            pltpu.VMEM((RWIN, CHR), jnp.int32),
            pltpu.VMEM((RWIN, CHR), jnp.int32),
            pltpu.VMEM((4 * CHR, D), jnp.float32),
            pltpu.VMEM_SHARED((NPAD, D), jnp.float32),
        ] + [pltpu.SemaphoreType.DMA] * 8,
        compiler_params=pltpu.CompilerParams(needs_layout_passes=False),
    )(_sc2_body)
    return f(hs1, ts2r, td2r)


def _sc3_body(hs2_hbm, hg_hbm, tsc_hbm, tdc_hbm, fsc_hbm, fdc_hbm, ex_hbm,
              acc2_out, accg_out,
              sidx, didx, exv, rows, acc_sh,
              g0, g1, g2, g3, ss0, ss1, ss2, ss3):
    cid = lax.axis_index("c")
    sid = lax.axis_index("s")
    gsems = (g0, g1, g2, g3)
    ssems = (ss0, ss1, ss2, ss3)

    _zero_acc_rows(rows, acc_sh, sid, gsems)
    plsc.subcore_barrier()

    @pl.when(cid == 0)
    def _():
        _ring_pass(hs2_hbm, tsc_hbm, tdc_hbm, None, sidx, didx, None, rows,
                   acc_sh, gsems, ssems, NWINR2, sid * NCHR2, False)

    @pl.when(cid == 1)
    def _():
        _ring_pass(hg_hbm, fsc_hbm, fdc_hbm, ex_hbm, sidx, didx, exv, rows,
                   acc_sh, gsems, ssems, NWINR2, sid * NCHR2, True)

    plsc.subcore_barrier()
    base = sid * TROW

    @pl.when(cid == 0)
    def _():
        pltpu.sync_copy(acc_sh.at[pl.ds(base, TROW)],
                        acc2_out.at[pl.ds(base, TROW)])

    @pl.when(cid == 1)
    def _():
        pltpu.sync_copy(acc_sh.at[pl.ds(base, TROW)],
                        accg_out.at[pl.ds(base, TROW)])


def _sc_stage3(hs2, hg, ts2r, td2r, fs2r, fd2r, ex2r):
    f = functools.partial(
        pl.kernel,
        out_type=[
            jax.ShapeDtypeStruct((NPAD, D), jnp.float32),
            jax.ShapeDtypeStruct((NPAD, D), jnp.float32),
        ],
        mesh=_SC_MESH,
        scratch_types=[
            pltpu.VMEM((RWIN, CHR), jnp.int32),
            pltpu.VMEM((RWIN, CHR), jnp.int32),
            pltpu.VMEM((RWIN, CHR), jnp.float32),
            pltpu.VMEM((4 * CHR, D), jnp.float32),
            pltpu.VMEM_SHARED((NPAD, D), jnp.float32),
        ] + [pltpu.SemaphoreType.DMA] * 8,
        compiler_params=pltpu.CompilerParams(needs_layout_passes=False),
    )(_sc3_body)
    return f(hs2, hg, ts2r, td2r, fs2r, fd2r, ex2r)


def kernel(topo_x, topo_edge_index, feat_x, feat_edge_index,
           W1, b1, W2, b2, Wg, att_l, att_r, Wd):
    pad_src = jnp.zeros((E2 - E,), jnp.int32)
    pad_dst = jnp.full((E2 - E,), N, jnp.int32)
    ts_p = jnp.concatenate([topo_edge_index[0], pad_src])
    td_p = jnp.concatenate([topo_edge_index[1], pad_dst])
    fs_p = jnp.concatenate([feat_edge_index[0], pad_src])
    fd_p = jnp.concatenate([feat_edge_index[1], pad_dst])
    td2 = td_p.reshape(ECH, CH)
    fs2 = fs_p.reshape(ECH, CH)
    fd2 = fd_p.reshape(ECH, CH)
    ts2r = ts_p.reshape(ECHR, CHR)
    td2r = td_p.reshape(ECHR, CHR)
    fs2r = fs_p.reshape(ECHR, CHR)
    fd2r = fd_p.reshape(ECHR, CHR)

    h1, hg, asrc, adst, mm = _tc_stage1(topo_x, feat_x, W1, Wg, att_l, att_r)

    m16 = jnp.zeros((16,), jnp.float32).at[:2].set(mm.reshape(2))
    padn = jnp.zeros((NPAD - N,), jnp.float32)
    asrc_p = jnp.concatenate([asrc.reshape(N), padn])
    adst_p = jnp.concatenate([adst.reshape(N), padn])
    deg_flat, s_flat, ex2 = _sc_stage1(td2, fs2, fd2, asrc_p, adst_p, m16)

    deg0 = deg_flat[:N].reshape(N, 1)
    deg1 = deg_flat[NPAD:NPAD + N].reshape(N, 1)
    dinv, hs1 = _tc_stage2(deg0, deg1, h1)

    acc1_flat = _sc_stage2(hs1, ts2r, td2r)
    acc1a = acc1_flat[:N]
    acc1b = acc1_flat[NPAD:NPAD + N]

    h2, hs2 = _tc_stage3(acc1a, acc1b, h1, dinv, b1, W2)

    ex2r = ex2.reshape(ECHR, CHR)
    acc2, accg = _sc_stage3(hs2, hg, ts2r, td2r, fs2r, fd2r, ex2r)

    s0 = s_flat[:N].reshape(N, 1)
    s1 = s_flat[NPAD:NPAD + N].reshape(N, 1)
    return _tc_stage4(acc2[:N], accg[:N],
                      h2, hg, dinv, s0, s1, asrc, adst, mm, b2, Wd)


# speedup vs baseline: 2.1472x; 1.5567x over previous
import functools

import jax
import jax.numpy as jnp
from jax import lax
from jax.experimental import pallas as pl
from jax.experimental.pallas import tpu as pltpu
from jax.experimental.pallas import tpu_sc as plsc

N = 10000
D = 128
BLK = 1000
GRID = N // BLK
NEG_INF = -3.0e38

E = 320000
NW = 32
NSUB = 16
CH = 80
E2 = 327680
ECH = E2 // CH
NCH = E2 // NW // CH
NCH2 = E2 // NSUB // CH
NPAD = 10240
TROW = NPAD // NSUB


def _lrelu(x):
    return jnp.where(x > 0, x, 0.2 * x)


def _tc1_body(xt_ref, xf_ref, w1_ref, wg_ref, al_ref, ar_ref,
              h1_ref, hg_ref, asrc_ref, adst_ref, m_ref):
    i = pl.program_id(0)
    h1 = jnp.dot(xt_ref[...], w1_ref[...], preferred_element_type=jnp.float32)
    h1_ref[...] = h1
    hg = jnp.dot(xf_ref[...], wg_ref[...], preferred_element_type=jnp.float32)
    hg_ref[...] = hg
    asrc = jnp.sum(hg * al_ref[...], axis=-1)
    adst = jnp.sum(hg * ar_ref[...], axis=-1)
    asrc_ref[...] = asrc[:, None]
    adst_ref[...] = adst[:, None]

    cur = jnp.where(i == 0, jnp.full((1, 2), NEG_INF, jnp.float32), m_ref[...])
    new = jnp.stack([jnp.max(asrc), jnp.max(adst)]).reshape(1, 2)
    m_ref[...] = jnp.maximum(cur, new)


def _tc_stage1(topo_x, feat_x, W1, Wg, att_l, att_r):
    return pl.pallas_call(
        _tc1_body,
        grid=(GRID,),
        in_specs=[
            pl.BlockSpec((BLK, D), lambda i: (i, 0)),
            pl.BlockSpec((BLK, D), lambda i: (i, 0)),
            pl.BlockSpec((D, D), lambda i: (0, 0)),
            pl.BlockSpec((D, D), lambda i: (0, 0)),
            pl.BlockSpec((1, D), lambda i: (0, 0)),
            pl.BlockSpec((1, D), lambda i: (0, 0)),
        ],
        out_specs=[
            pl.BlockSpec((BLK, D), lambda i: (i, 0)),
            pl.BlockSpec((BLK, D), lambda i: (i, 0)),
            pl.BlockSpec((BLK, 1), lambda i: (i, 0)),
            pl.BlockSpec((BLK, 1), lambda i: (i, 0)),
            pl.BlockSpec((1, 2), lambda i: (0, 0)),
        ],
        out_shape=[
            jax.ShapeDtypeStruct((N, D), jnp.float32),
            jax.ShapeDtypeStruct((N, D), jnp.float32),
            jax.ShapeDtypeStruct((N, 1), jnp.float32),
            jax.ShapeDtypeStruct((N, 1), jnp.float32),
            jax.ShapeDtypeStruct((1, 2), jnp.float32),
        ],
    )(topo_x, feat_x, W1, Wg, att_l.reshape(1, D), att_r.reshape(1, D))


def _tc2_body(d0_ref, d1_ref, h1_ref, dinv_ref, hs1_ref):
    deg = d0_ref[:, 0] + d1_ref[:, 0] + 1.0
    dinv = lax.rsqrt(deg)
    dinv_ref[...] = dinv[:, None]
    hs1_ref[...] = h1_ref[...] * dinv[:, None]


def _tc_stage2(deg0, deg1, h1):
    return pl.pallas_call(
        _tc2_body,
        grid=(GRID,),
        in_specs=[
            pl.BlockSpec((BLK, 1), lambda i: (i, 0)),
            pl.BlockSpec((BLK, 1), lambda i: (i, 0)),
            pl.BlockSpec((BLK, D), lambda i: (i, 0)),
        ],
        out_specs=[
            pl.BlockSpec((BLK, 1), lambda i: (i, 0)),
            pl.BlockSpec((BLK, D), lambda i: (i, 0)),
        ],
        out_shape=[
            jax.ShapeDtypeStruct((N, 1), jnp.float32),
            jax.ShapeDtypeStruct((N, D), jnp.float32),
        ],
    )(deg0, deg1, h1)


def _tc3_body(a0_ref, a1_ref, h1_ref, dinv_ref, b1_ref, w2_ref,
              h2_ref, hs2_ref):
    di = dinv_ref[:, 0]
    a = a0_ref[...] + a1_ref[...]
    x2 = jnp.maximum(
        di[:, None] * a + (di * di)[:, None] * h1_ref[...] + b1_ref[...], 0.0)
    h2 = jnp.dot(x2, w2_ref[...], preferred_element_type=jnp.float32)
    h2_ref[...] = h2
    hs2_ref[...] = h2 * di[:, None]


def _tc_stage3(acc1a, acc1b, h1, dinv, b1, W2):
    return pl.pallas_call(
        _tc3_body,
        grid=(GRID,),
        in_specs=[
            pl.BlockSpec((BLK, D), lambda i: (i, 0)),
            pl.BlockSpec((BLK, D), lambda i: (i, 0)),
            pl.BlockSpec((BLK, D), lambda i: (i, 0)),
            pl.BlockSpec((BLK, 1), lambda i: (i, 0)),
            pl.BlockSpec((1, D), lambda i: (0, 0)),
            pl.BlockSpec((D, D), lambda i: (0, 0)),
        ],
        out_specs=[
            pl.BlockSpec((BLK, D), lambda i: (i, 0)),
            pl.BlockSpec((BLK, D), lambda i: (i, 0)),
        ],
        out_shape=[
            jax.ShapeDtypeStruct((N, D), jnp.float32),
            jax.ShapeDtypeStruct((N, D), jnp.float32),
        ],
    )(acc1a, acc1b, h1, dinv, b1.reshape(1, D), W2)


def _tc4_body(a2_ref, ag_ref, h2_ref, hg_ref, dinv_ref,
              s0_ref, s1_ref, asrc_ref, adst_ref, m_ref, b2_ref, wd_ref,
              res_ref):
    di = dinv_ref[:, 0]
    topo_z = (di[:, None] * a2_ref[...] + (di * di)[:, None] * h2_ref[...]
              + b2_ref[...])
    mv = m_ref[...]
    M = _lrelu(mv[0, 0] + mv[0, 1])
    ex_self = jnp.exp(_lrelu(asrc_ref[:, 0] + adst_ref[:, 0]) - M)
    s = s0_ref[:, 0] + s1_ref[:, 0] + ex_self
    feat_z = (ag_ref[...] + ex_self[:, None] * hg_ref[...]) / (
        s[:, None] + 1e-16)
    fzw = jnp.dot(feat_z, wd_ref[...], preferred_element_type=jnp.float32)
    res = jax.nn.sigmoid(jnp.sum(topo_z * fzw, axis=-1))
    res_ref[...] = res[:, None]


def _tc_stage4(a2, ag, h2, hg, dinv, s0, s1, asrc, adst, mm, b2, Wd):
    res = pl.pallas_call(
        _tc4_body,
        grid=(GRID,),
        in_specs=[
            pl.BlockSpec((BLK, D), lambda i: (i, 0)),
            pl.BlockSpec((BLK, D), lambda i: (i, 0)),
            pl.BlockSpec((BLK, D), lambda i: (i, 0)),
            pl.BlockSpec((BLK, D), lambda i: (i, 0)),
            pl.BlockSpec((BLK, 1), lambda i: (i, 0)),
            pl.BlockSpec((BLK, 1), lambda i: (i, 0)),
            pl.BlockSpec((BLK, 1), lambda i: (i, 0)),
            pl.BlockSpec((BLK, 1), lambda i: (i, 0)),
            pl.BlockSpec((BLK, 1), lambda i: (i, 0)),
            pl.BlockSpec((1, 2), lambda i: (0, 0)),
            pl.BlockSpec((1, D), lambda i: (0, 0)),
            pl.BlockSpec((D, D), lambda i: (0, 0)),
        ],
        out_specs=[pl.BlockSpec((BLK, 1), lambda i: (i, 0))],
        out_shape=[jax.ShapeDtypeStruct((N, 1), jnp.float32)],
    )(a2, ag, h2, hg, dinv, s0, s1, asrc, adst, mm,
      b2.reshape(1, D), Wd)[0]
    return res.reshape(N)


_SC_MESH = plsc.VectorSubcoreMesh(core_axis_name="c", subcore_axis_name="s")


def _sc1_body(td_hbm, fs_hbm, fd_hbm, asrc_hbm, adst_hbm, m_hbm,
              deg_out, s_out, ex_out,
              asrc_v, adst_v, tdi, fsi, fdi, exv, ones_v, zv, m_v,
              deg_sh, s_sh):
    cid = lax.axis_index("c")
    sid = lax.axis_index("s")
    wid = cid * NSUB + sid

    def _zbody(i, _):
        zv[pl.ds(i * 16, 16)] = jnp.zeros((16,), jnp.float32)
        return _
    lax.fori_loop(0, TROW // 16, _zbody, None)
    pltpu.sync_copy(zv, deg_sh.at[pl.ds(sid * TROW, TROW)])
    pltpu.sync_copy(zv, s_sh.at[pl.ds(sid * TROW, TROW)])

    def _obody(i, _):
        ones_v[pl.ds(i * 16, 16)] = jnp.ones((16,), jnp.float32)
        return _
    lax.fori_loop(0, CH // 16, _obody, None)

    pltpu.sync_copy(asrc_hbm, asrc_v)
    pltpu.sync_copy(adst_hbm, adst_v)
    pltpu.sync_copy(m_hbm, m_v)
    pltpu.sync_copy(td_hbm.at[pl.ds(wid * NCH, NCH)], tdi)
    pltpu.sync_copy(fs_hbm.at[pl.ds(wid * NCH, NCH)], fsi)
    pltpu.sync_copy(fd_hbm.at[pl.ds(wid * NCH, NCH)], fdi)
    plsc.subcore_barrier()

    mv = m_v[pl.ds(0, 16)]
    M = _lrelu(mv[0] + mv[1])

    def _chunk(j, _):
        for r in range(CH // 16):
            sv = fsi[j, pl.ds(r * 16, 16)]
            dv = fdi[j, pl.ds(r * 16, 16)]
            av = plsc.load_gather(asrc_v, [sv])
            bv = plsc.load_gather(adst_v, [dv])
            ex = jnp.exp(_lrelu(av + bv) - M)
            exv[j, pl.ds(r * 16, 16)] = ex
        pltpu.sync_copy(exv.at[j], s_sh.at[fdi.at[j]], add=True)
        pltpu.sync_copy(ones_v, deg_sh.at[tdi.at[j]], add=True)
        return _
    lax.fori_loop(0, NCH, _chunk, None)

    pltpu.sync_copy(exv, ex_out.at[pl.ds(wid * NCH, NCH)])
    plsc.subcore_barrier()

    base = cid * NPAD + sid * TROW
    pltpu.sync_copy(deg_sh.at[pl.ds(sid * TROW, TROW)],
                    deg_out.at[pl.ds(base, TROW)])
    pltpu.sync_copy(s_sh.at[pl.ds(sid * TROW, TROW)],
                    s_out.at[pl.ds(base, TROW)])


def _sc_stage1(td2, fs2, fd2, asrc, adst, m16):
    f = functools.partial(
        pl.kernel,
        out_type=[
            jax.ShapeDtypeStruct((2 * NPAD,), jnp.float32),
            jax.ShapeDtypeStruct((2 * NPAD,), jnp.float32),
            jax.ShapeDtypeStruct((ECH, CH), jnp.float32),
        ],
        mesh=_SC_MESH,
        scratch_types=[
            pltpu.VMEM((NPAD,), jnp.float32),
            pltpu.VMEM((NPAD,), jnp.float32),
            pltpu.VMEM((NCH, CH), jnp.int32),
            pltpu.VMEM((NCH, CH), jnp.int32),
            pltpu.VMEM((NCH, CH), jnp.int32),
            pltpu.VMEM((NCH, CH), jnp.float32),
            pltpu.VMEM((CH,), jnp.float32),
            pltpu.VMEM((TROW,), jnp.float32),
            pltpu.VMEM((16,), jnp.float32),
            pltpu.VMEM_SHARED((NPAD,), jnp.float32),
            pltpu.VMEM_SHARED((NPAD,), jnp.float32),
        ],
        compiler_params=pltpu.CompilerParams(needs_layout_passes=False),
    )(_sc1_body)
    return f(td2, fs2, fd2, asrc, adst, m16)


CHR = 40
ECHR = E2 // CHR
NCHR = E2 // NW // CHR
NCHR2 = E2 // NSUB // CHR
RWIN = 64
NWINR = NCHR // RWIN
NWINR2 = NCHR2 // RWIN


def _zero_acc_rows(rows, acc_sh, sid, sems):
    def _zbody(i, _):
        for c in range(D // 16):
            rows[i, pl.ds(c * 16, 16)] = jnp.zeros((16,), jnp.float32)
        return _
    lax.fori_loop(0, CHR, _zbody, None)
    nk = TROW // CHR
    for k in range(nk):
        pltpu.async_copy(rows.at[pl.ds(0, CHR)],
                         acc_sh.at[pl.ds(sid * TROW + k * CHR, CHR)],
                         sems[k % 4])
    for k in range(nk):
        pltpu.make_async_copy(rows.at[pl.ds(0, CHR)],
                              acc_sh.at[pl.ds(sid * TROW + k * CHR, CHR)],
                              sems[k % 4]).wait()


def _ring_pass(table, srcc, dstc, exs, sidx, didx, exv, graw, frows, acc_sh,
               gsems, ssems, nwin, chunk_base, scale):
    def _startg(j, b):
        pltpu.async_copy(table.at[sidx.at[j]], graw.at[b], gsems[b])

    def _waitg(j, b):
        pltpu.make_async_copy(table.at[sidx.at[j]], graw.at[b],
                              gsems[b]).wait()

    def _starts(j, b2):
        pltpu.async_copy(frows.at[pl.ds(b2 * CHR, CHR)],
                         acc_sh.at[didx.at[j]], ssems[b2], add=True)

    def _waits(j, b2):
        pltpu.make_async_copy(frows.at[pl.ds(b2 * CHR, CHR)],
                              acc_sh.at[didx.at[j]], ssems[b2]).wait()

    def _conv(j, b, b2):
        for off, lo in ((0, 0), (16, 0), (24, 8)):
            ev = exv[j, pl.ds(off, 16)] if scale else None
            for i in range(lo, 16):
                r = off + i
                sc = ev[i] if scale else None
                for g in range(4):
                    iv = graw[b, r, pl.ds(g * 16, 16)]
                    bb = plsc.bitcast(iv, jnp.bfloat16)
                    va, vb = plsc.unpack(
                        bb, format=plsc.PackFormat.INTERLEAVED,
                        preferred_element_type=jnp.float32)
                    if scale:
                        va = va * sc
                        vb = vb * sc
                    frows[b2 * CHR + r, pl.ds(g * 32, 16)] = va
                    frows[b2 * CHR + r, pl.ds(g * 32 + 16, 16)] = vb

    def _window(w, _):
        wrow = chunk_base + w * RWIN
        pltpu.sync_copy(srcc.at[pl.ds(wrow, RWIN)], sidx)
        pltpu.sync_copy(dstc.at[pl.ds(wrow, RWIN)], didx)
        if scale:
            pltpu.sync_copy(exs.at[pl.ds(wrow, RWIN)], exv)
        for k in range(4):
            _startg(k, k)

        def _ring(j0, __):
            for b in range(4):
                j = 4 * j0 + b
                b2 = b % 2
                _waitg(j, b)

                @pl.when(j >= 2)
                def _():
                    _waits(j - 2, b2)

                _conv(j, b, b2)
                _starts(j, b2)

                @pl.when(j + 4 < RWIN)
                def _():
                    _startg(j + 4, b)
            return __
        lax.fori_loop(0, RWIN // 4, _ring, None)
        _waits(RWIN - 2, 0)
        _waits(RWIN - 1, 1)
        return _
    lax.fori_loop(0, nwin, _window, None)


def _sc2_body(hs_hbm, sc_hbm, dc_hbm, out_hbm,
              sidx, didx, graw, frows, acc_sh,
              g0, g1, g2, g3, ss0, ss1):
    cid = lax.axis_index("c")
    sid = lax.axis_index("s")
    wid = sid * 2 + cid

    _zero_acc_rows(frows, acc_sh, sid, (g0, g1, g2, g3))
    plsc.subcore_barrier()
    _ring_pass(hs_hbm, sc_hbm, dc_hbm, None, sidx, didx, None, graw, frows,
               acc_sh, (g0, g1, g2, g3), (ss0, ss1),
               NWINR, wid * NCHR, False)
    plsc.subcore_barrier()
    base = cid * NPAD + sid * TROW
    pltpu.sync_copy(acc_sh.at[pl.ds(sid * TROW, TROW)],
                    out_hbm.at[pl.ds(base, TROW)])


def _sc_stage2(hs1p, ts2r, td2r):
    f = functools.partial(
        pl.kernel,
        out_type=jax.ShapeDtypeStruct((2 * NPAD, D), jnp.float32),
        mesh=_SC_MESH,
        scratch_types=[
            pltpu.VMEM((RWIN, CHR), jnp.int32),
            pltpu.VMEM((RWIN, CHR), jnp.int32),
            pltpu.VMEM((4, CHR, D // 2), jnp.int32),
            pltpu.VMEM((2 * CHR, D), jnp.float32),
            pltpu.VMEM_SHARED((NPAD, D), jnp.float32),
        ] + [pltpu.SemaphoreType.DMA] * 6,
        compiler_params=pltpu.CompilerParams(needs_layout_passes=False,
                                             use_tc_tiling_on_sc=False),
    )(_sc2_body)
    return f(hs1p, ts2r, td2r)


def _sc3_body(hs2_hbm, hg_hbm, tsc_hbm, tdc_hbm, fsc_hbm, fdc_hbm, ex_hbm,
              acc2_out, accg_out,
              sidx, didx, exv, graw, frows, acc_sh,
              g0, g1, g2, g3, ss0, ss1):
    cid = lax.axis_index("c")
    sid = lax.axis_index("s")
    gsems = (g0, g1, g2, g3)
    ssems = (ss0, ss1)

    _zero_acc_rows(frows, acc_sh, sid, gsems)
    plsc.subcore_barrier()

    @pl.when(cid == 0)
    def _():
        _ring_pass(hs2_hbm, tsc_hbm, tdc_hbm, None, sidx, didx, None, graw,
                   frows, acc_sh, gsems, ssems, NWINR2, sid * NCHR2, False)

    @pl.when(cid == 1)
    def _():
        _ring_pass(hg_hbm, fsc_hbm, fdc_hbm, ex_hbm, sidx, didx, exv, graw,
                   frows, acc_sh, gsems, ssems, NWINR2, sid * NCHR2, True)

    plsc.subcore_barrier()
    base = sid * TROW

    @pl.when(cid == 0)
    def _():
        pltpu.sync_copy(acc_sh.at[pl.ds(base, TROW)],
                        acc2_out.at[pl.ds(base, TROW)])

    @pl.when(cid == 1)
    def _():
        pltpu.sync_copy(acc_sh.at[pl.ds(base, TROW)],
                        accg_out.at[pl.ds(base, TROW)])


def _sc_stage3(hs2p, hgp, ts2r, td2r, fs2r, fd2r, ex2r):
    f = functools.partial(
        pl.kernel,
        out_type=[
            jax.ShapeDtypeStruct((NPAD, D), jnp.float32),
            jax.ShapeDtypeStruct((NPAD, D), jnp.float32),
        ],
        mesh=_SC_MESH,
        scratch_types=[
            pltpu.VMEM((RWIN, CHR), jnp.int32),
            pltpu.VMEM((RWIN, CHR), jnp.int32),
            pltpu.VMEM((RWIN, CHR), jnp.float32),
            pltpu.VMEM((4, CHR, D // 2), jnp.int32),
            pltpu.VMEM((2 * CHR, D), jnp.float32),
            pltpu.VMEM_SHARED((NPAD, D), jnp.float32),
        ] + [pltpu.SemaphoreType.DMA] * 6,
        compiler_params=pltpu.CompilerParams(needs_layout_passes=False,
                                             use_tc_tiling_on_sc=False),
    )(_sc3_body)
    return f(hs2p, hgp, ts2r, td2r, fs2r, fd2r, ex2r)


def _pack_bf16(x):
    xb = x.astype(jnp.bfloat16).reshape(N, 4, 2, 16)
    st = jnp.stack([xb[:, :, 0, :], xb[:, :, 1, :]], axis=-1)
    return jax.lax.bitcast_convert_type(st, jnp.int32).reshape(N, D // 2)


def kernel(topo_x, topo_edge_index, feat_x, feat_edge_index,
           W1, b1, W2, b2, Wg, att_l, att_r, Wd):
    pad_src = jnp.zeros((E2 - E,), jnp.int32)
    pad_dst = jnp.full((E2 - E,), N, jnp.int32)
    ts_p = jnp.concatenate([topo_edge_index[0], pad_src])
    td_p = jnp.concatenate([topo_edge_index[1], pad_dst])
    fs_p = jnp.concatenate([feat_edge_index[0], pad_src])
    fd_p = jnp.concatenate([feat_edge_index[1], pad_dst])
    td2 = td_p.reshape(ECH, CH)
    fs2 = fs_p.reshape(ECH, CH)
    fd2 = fd_p.reshape(ECH, CH)
    ts2r = ts_p.reshape(ECHR, CHR)
    td2r = td_p.reshape(ECHR, CHR)
    fs2r = fs_p.reshape(ECHR, CHR)
    fd2r = fd_p.reshape(ECHR, CHR)

    h1, hg, asrc, adst, mm = _tc_stage1(topo_x, feat_x, W1, Wg, att_l, att_r)

    m16 = jnp.zeros((16,), jnp.float32).at[:2].set(mm.reshape(2))
    padn = jnp.zeros((NPAD - N,), jnp.float32)
    asrc_p = jnp.concatenate([asrc.reshape(N), padn])
    adst_p = jnp.concatenate([adst.reshape(N), padn])
    deg_flat, s_flat, ex2 = _sc_stage1(td2, fs2, fd2, asrc_p, adst_p, m16)

    deg0 = deg_flat[:N].reshape(N, 1)
    deg1 = deg_flat[NPAD:NPAD + N].reshape(N, 1)
    dinv, hs1 = _tc_stage2(deg0, deg1, h1)

    acc1_flat = _sc_stage2(_pack_bf16(hs1), ts2r, td2r)
    acc1a = acc1_flat[:N]
    acc1b = acc1_flat[NPAD:NPAD + N]

    h2, hs2 = _tc_stage3(acc1a, acc1b, h1, dinv, b1, W2)

    ex2r = ex2.reshape(ECHR, CHR)
    acc2, accg = _sc_stage3(_pack_bf16(hs2), _pack_bf16(hg),
                            ts2r, td2r, fs2r, fd2r, ex2r)

    s0 = s_flat[:N].reshape(N, 1)
    s1 = s_flat[NPAD:NPAD + N].reshape(N, 1)
    return _tc_stage4(acc2[:N], accg[:N],
                      h2, hg, dinv, s0, s1, asrc, adst, mm, b2, Wd)
